# baseline (speedup 1.0000x reference)
import functools

import jax
import jax.numpy as jnp
from jax.experimental import pallas as pl
from jax.experimental.pallas import tpu as pltpu

_DT = 0.1
_BF16 = jnp.bfloat16

_N_STEPS = 8


def _unet_kernel(x_ref, enc_w_ref, bias_ref,
                 mid_w_ref, dec_w_ref, out_w_ref,
                 o_ref, h1b_ref, pooled_ref, ub_ref, acc_ref,
                 dec_buf_ref, dec_sem,
                 *, N, H, W, hidden, dt):
    f32 = jnp.float32
    HW = H * W
    L = N * HW
    step = pl.program_id(0)

    col = jax.lax.broadcasted_iota(jnp.int32, (1, L), 1)
    w_pos = col % W
    h_pos = (col // W) % H

    def shift(v, s):
        k = (-s) % L
        return v if k == 0 else pltpu.roll(v, k, 1)

    def tap_piece(vb, tap, dil, mask_base):
        kh, kw = tap // 3, tap % 3
        dh, dw = (kh - 1) * dil, (kw - 1) * dil
        t = shift(vb, dh * W + dw)
        if dh != 0 or dw != 0:
            valid = ((h_pos + dh >= 0) & (h_pos + dh < H) &
                     (w_pos + dw >= 0) & (w_pos + dw < W))
            t = t * valid.astype(_BF16)
        return t

    def bias_col(c):
        return jnp.broadcast_to(bias_ref[:, c:c + 1], (hidden, L))

    @pl.when(step == 0)
    def _enc_pool():
        xb = x_ref[...].astype(_BF16)
        pieces = [tap_piece(xb, t, 1, 0) for t in range(9)]
        pieces.append(jnp.ones((1, L), _BF16))
        stk = jnp.concatenate(pieces, axis=0)
        h1 = jnp.maximum(
            jnp.dot(enc_w_ref[...], stk, preferred_element_type=f32), 0.0)
        h1b = h1.astype(_BF16)
        h1b_ref[...] = h1b

        w_even = (col % 2) == 0
        h_even = ((col // W) % 2) == 0
        p_w = jnp.where(w_even,
                        jnp.maximum(h1b, shift(h1b, 1)),
                        jnp.maximum(h1b, shift(h1b, -1)))
        pooled_ref[...] = jnp.where(h_even,
                                    jnp.maximum(p_w, shift(p_w, W)),
                                    jnp.maximum(p_w, shift(p_w, -W)))
        acc_ref[...] = bias_col(0)
        pltpu.make_async_copy(dec_w_ref, dec_buf_ref, dec_sem).start()

    for s in range(3):
        @pl.when(step == 1 + s)
        def _mid_block(s=s):
            pooled = pooled_ref[...]
            acc = acc_ref[...]
            for j in range(3):
                tap = 3 * s + j
                pc = tap_piece(pooled, tap, 2, 9)
                acc = acc + jnp.dot(mid_w_ref[:, j * hidden:(j + 1) * hidden],
                                    pc, preferred_element_type=f32)
            acc_ref[...] = acc

    @pl.when(step == 3)
    def _mid_done():
        ub_ref[...] = jnp.maximum(acc_ref[...], 0.0).astype(_BF16)
        acc_ref[...] = bias_col(1)
        pltpu.make_async_copy(dec_w_ref, dec_buf_ref, dec_sem).wait()

    for s in range(3):
        @pl.when(step == 4 + s)
        def _dec_taps(s=s):
            h1b = h1b_ref[...]
            ub = ub_ref[...]
            acc = acc_ref[...]
            for j in range(3):
                tap = 3 * s + j
                pc_h1 = tap_piece(h1b, tap, 1, 0)
                pc_u = tap_piece(ub, tap, 1, 0)
                lo = 2 * tap * hidden
                acc = (acc
                       + jnp.dot(dec_buf_ref[:, lo:lo + hidden], pc_h1,
                                 preferred_element_type=f32)
                       + jnp.dot(dec_buf_ref[:, lo + hidden:lo + 2 * hidden],
                                 pc_u, preferred_element_type=f32))
            acc_ref[...] = acc

    @pl.when(step == _N_STEPS - 1)
    def _out_mask():
        d = jnp.maximum(acc_ref[...], 0.0).astype(_BF16)
        y = jnp.dot(out_w_ref[...],
                    jnp.concatenate([d, jnp.ones((1, L), _BF16)], axis=0),
                    preferred_element_type=f32)
        x = x_ref[...]
        x0 = x[0:1, 0:HW]
        z0 = x[1:2, 0:HW]
        t1 = x[2:3, 0:HW] + dt
        m = jnp.where(x0 * x0 + z0 * z0 <= t1 * t1, 1.0, 0.0)
        if N > 1:
            m = jnp.concatenate([m] * N, axis=1)
        o_ref[...] = (y * m).astype(o_ref.dtype)


def _taps_only(w, cin_pad=None, dtype=_BF16):
    cout, cin, kh, kw = w.shape
    taps = [w[:, :, i, j] for i in range(kh) for j in range(kw)]
    if cin_pad is not None and cin_pad != cin:
        taps = [jnp.pad(t, ((0, 0), (0, cin_pad - cin))) for t in taps]
    return jnp.concatenate(taps, axis=1).astype(dtype)


def kernel(enc_w, enc_b, mid_w, mid_b, dec_w, dec_b, out_w, out_b, x):
    N, ci, H, W = x.shape
    hidden = enc_w.shape[0]
    co = out_w.shape[0]
    HW, L = H * W, N * H * W
    ci_p = max(8, ((ci + 7) // 8) * 8)
    co_p = max(8, ((co + 7) // 8) * 8)

    if N == 1:
        x_cl = x.reshape(ci, L)
    else:
        x_cl = jnp.transpose(x.reshape(N, ci, HW), (1, 0, 2)).reshape(ci, L)
    if ci_p != ci:
        x_cl = jnp.pad(x_cl, ((0, ci_p - ci), (0, 0)))

    enc_ws = jnp.concatenate(
        [_taps_only(enc_w, ci_p), enc_b.reshape(hidden, 1).astype(_BF16)],
        axis=1)
    mid_ws = _taps_only(mid_w)
    dec_ws = _taps_only(dec_w)
    out_ws = jnp.concatenate(
        [out_w.reshape(co, hidden), out_b.reshape(co, 1)], axis=1)
    if co_p != co:
        out_ws = jnp.pad(out_ws, ((0, co_p - co), (0, 0)))
    out_ws = out_ws.astype(_BF16)

    biases = jnp.pad(jnp.stack([mid_b, dec_b], axis=1),
                     ((0, 0), (0, 126)))

    kfn = functools.partial(_unet_kernel, N=N, H=H, W=W,
                            hidden=hidden, dt=float(_DT))

    flops = 2 * L * (hidden * (9 * ci_p + 1) + hidden * (9 * hidden + 1)
                     + hidden * (18 * hidden + 1) + co_p * (hidden + 1))
    bytes_accessed = int(4 * (x_cl.size + biases.size + co_p * L)
                         + 2 * (enc_ws.size + mid_ws.size + dec_ws.size
                                + out_ws.size))

    out = pl.pallas_call(
        kfn,
        out_shape=jax.ShapeDtypeStruct((co_p, L), jnp.float32),
        grid=(_N_STEPS,),
        in_specs=[
            pl.BlockSpec((ci_p, L), lambda i: (0, 0)),
            pl.BlockSpec(enc_ws.shape, lambda i: (0, 0)),
            pl.BlockSpec(biases.shape, lambda i: (0, 0)),
            pl.BlockSpec((hidden, 3 * hidden),
                         lambda i: (0, jnp.clip(i - 1, 0, 2))),
            pl.BlockSpec(memory_space=pltpu.MemorySpace.HBM),
            pl.BlockSpec(out_ws.shape, lambda i: (0, 0)),
        ],
        out_specs=pl.BlockSpec((co_p, L), lambda i: (0, 0)),
        scratch_shapes=[
            pltpu.VMEM((hidden, L), _BF16),
            pltpu.VMEM((hidden, L), _BF16),
            pltpu.VMEM((hidden, L), _BF16),
            pltpu.VMEM((hidden, L), jnp.float32),
            pltpu.VMEM((hidden, 18 * hidden), _BF16),
            pltpu.SemaphoreType.DMA,
        ],
        compiler_params=pltpu.CompilerParams(
            dimension_semantics=("arbitrary",)),
        cost_estimate=pl.CostEstimate(flops=flops, transcendentals=0,
                                      bytes_accessed=bytes_accessed),
    )(x_cl, enc_ws, biases, mid_ws, dec_ws, out_ws)

    if N == 1:
        return out[:co].reshape(1, co, H, W)
    return out.reshape(co_p, N, H, W).transpose(1, 0, 2, 3)[:, :co]


# speedup vs baseline: 1.0094x; 1.0094x over previous
import functools

import jax
import jax.numpy as jnp
from jax.experimental import pallas as pl
from jax.experimental.pallas import tpu as pltpu

_DT = 0.1
_BF16 = jnp.bfloat16

_N_STEPS = 8


def _unet_kernel(x_ref, enc_w_ref, bias_ref,
                 mid_w_ref, dec_w_ref, out_w_ref,
                 o_ref, h1b_ref, pooled_ref, ub_ref, acc_ref,
                 *, N, H, W, hidden, dt):
    f32 = jnp.float32
    HW = H * W
    L = N * HW
    step = pl.program_id(0)

    col = jax.lax.broadcasted_iota(jnp.int32, (1, L), 1)
    w_pos = col % W
    h_pos = (col // W) % H

    def shift(v, s):
        k = (-s) % L
        return v if k == 0 else pltpu.roll(v, k, 1)

    def tap_piece(vb, tap, dil, mask_base):
        kh, kw = tap // 3, tap % 3
        dh, dw = (kh - 1) * dil, (kw - 1) * dil
        t = shift(vb, dh * W + dw)
        if dh != 0 or dw != 0:
            valid = ((h_pos + dh >= 0) & (h_pos + dh < H) &
                     (w_pos + dw >= 0) & (w_pos + dw < W))
            t = t * valid.astype(_BF16)
        return t

    def bias_col(c):
        return jnp.broadcast_to(bias_ref[:, c:c + 1], (hidden, L))

    @pl.when(step == 0)
    def _enc_pool():
        xb = x_ref[...].astype(_BF16)
        pieces = [tap_piece(xb, t, 1, 0) for t in range(9)]
        pieces.append(jnp.ones((1, L), _BF16))
        stk = jnp.concatenate(pieces, axis=0)
        h1 = jnp.maximum(
            jnp.dot(enc_w_ref[...], stk, preferred_element_type=f32), 0.0)
        h1b = h1.astype(_BF16)
        h1b_ref[...] = h1b

        w_even = (col % 2) == 0
        h_even = ((col // W) % 2) == 0
        p_w = jnp.where(w_even,
                        jnp.maximum(h1b, shift(h1b, 1)),
                        jnp.maximum(h1b, shift(h1b, -1)))
        pooled_ref[...] = jnp.where(h_even,
                                    jnp.maximum(p_w, shift(p_w, W)),
                                    jnp.maximum(p_w, shift(p_w, -W)))
        acc_ref[...] = bias_col(0)

    for s in range(3):
        @pl.when(step == 1 + s)
        def _mid_block(s=s):
            pooled = pooled_ref[...]
            acc = acc_ref[...]
            for j in range(3):
                tap = 3 * s + j
                pc = tap_piece(pooled, tap, 2, 9)
                acc = acc + jnp.dot(mid_w_ref[:, j * hidden:(j + 1) * hidden],
                                    pc, preferred_element_type=f32)
            acc_ref[...] = acc

    @pl.when(step == 3)
    def _mid_done():
        ub_ref[...] = jnp.maximum(acc_ref[...], 0.0).astype(_BF16)
        acc_ref[...] = bias_col(1)

    for s in range(3):
        @pl.when(step == 4 + s)
        def _dec_taps(s=s):
            h1b = h1b_ref[...]
            ub = ub_ref[...]
            acc = acc_ref[...]
            for j in range(3):
                tap = 3 * s + j
                pc_h1 = tap_piece(h1b, tap, 1, 0)
                pc_u = tap_piece(ub, tap, 1, 0)
                lo = 2 * j * hidden
                acc = (acc
                       + jnp.dot(dec_w_ref[:, lo:lo + hidden], pc_h1,
                                 preferred_element_type=f32)
                       + jnp.dot(dec_w_ref[:, lo + hidden:lo + 2 * hidden],
                                 pc_u, preferred_element_type=f32))
            acc_ref[...] = acc

    @pl.when(step == _N_STEPS - 1)
    def _out_mask():
        d = jnp.maximum(acc_ref[...], 0.0).astype(_BF16)
        y = jnp.dot(out_w_ref[...],
                    jnp.concatenate([d, jnp.ones((1, L), _BF16)], axis=0),
                    preferred_element_type=f32)
        x = x_ref[...]
        x0 = x[0:1, 0:HW]
        z0 = x[1:2, 0:HW]
        t1 = x[2:3, 0:HW] + dt
        m = jnp.where(x0 * x0 + z0 * z0 <= t1 * t1, 1.0, 0.0)
        if N > 1:
            m = jnp.concatenate([m] * N, axis=1)
        o_ref[...] = (y * m).astype(o_ref.dtype)


def _taps_only(w, cin_pad=None, dtype=_BF16):
    cout, cin, kh, kw = w.shape
    taps = [w[:, :, i, j] for i in range(kh) for j in range(kw)]
    if cin_pad is not None and cin_pad != cin:
        taps = [jnp.pad(t, ((0, 0), (0, cin_pad - cin))) for t in taps]
    return jnp.concatenate(taps, axis=1).astype(dtype)


def kernel(enc_w, enc_b, mid_w, mid_b, dec_w, dec_b, out_w, out_b, x):
    N, ci, H, W = x.shape
    hidden = enc_w.shape[0]
    co = out_w.shape[0]
    HW, L = H * W, N * H * W
    ci_p = max(8, ((ci + 7) // 8) * 8)
    co_p = max(8, ((co + 7) // 8) * 8)

    if N == 1:
        x_cl = x.reshape(ci, L)
    else:
        x_cl = jnp.transpose(x.reshape(N, ci, HW), (1, 0, 2)).reshape(ci, L)
    if ci_p != ci:
        x_cl = jnp.pad(x_cl, ((0, ci_p - ci), (0, 0)))

    enc_ws = jnp.concatenate(
        [_taps_only(enc_w, ci_p), enc_b.reshape(hidden, 1).astype(_BF16)],
        axis=1)
    mid_ws = _taps_only(mid_w)
    dec_ws = _taps_only(dec_w)
    out_ws = jnp.concatenate(
        [out_w.reshape(co, hidden), out_b.reshape(co, 1)], axis=1)
    if co_p != co:
        out_ws = jnp.pad(out_ws, ((0, co_p - co), (0, 0)))
    out_ws = out_ws.astype(_BF16)

    biases = jnp.pad(jnp.stack([mid_b, dec_b], axis=1),
                     ((0, 0), (0, 126)))

    kfn = functools.partial(_unet_kernel, N=N, H=H, W=W,
                            hidden=hidden, dt=float(_DT))

    flops = 2 * L * (hidden * (9 * ci_p + 1) + hidden * (9 * hidden + 1)
                     + hidden * (18 * hidden + 1) + co_p * (hidden + 1))
    bytes_accessed = int(4 * (x_cl.size + biases.size + co_p * L)
                         + 2 * (enc_ws.size + mid_ws.size + dec_ws.size
                                + out_ws.size))

    out = pl.pallas_call(
        kfn,
        out_shape=jax.ShapeDtypeStruct((co_p, L), jnp.float32),
        grid=(_N_STEPS,),
        in_specs=[
            pl.BlockSpec((ci_p, L), lambda i: (0, 0)),
            pl.BlockSpec(enc_ws.shape, lambda i: (0, 0)),
            pl.BlockSpec(biases.shape, lambda i: (0, 0)),
            pl.BlockSpec((hidden, 3 * hidden),
                         lambda i: (0, jnp.clip(i - 1, 0, 2))),
            pl.BlockSpec((hidden, 6 * hidden),
                         lambda i: (0, jnp.clip(i - 4, 0, 2))),
            pl.BlockSpec(out_ws.shape, lambda i: (0, 0)),
        ],
        out_specs=pl.BlockSpec((co_p, L), lambda i: (0, 0)),
        scratch_shapes=[
            pltpu.VMEM((hidden, L), _BF16),
            pltpu.VMEM((hidden, L), _BF16),
            pltpu.VMEM((hidden, L), _BF16),
            pltpu.VMEM((hidden, L), jnp.float32),
        ],
        compiler_params=pltpu.CompilerParams(
            dimension_semantics=("arbitrary",)),
        cost_estimate=pl.CostEstimate(flops=flops, transcendentals=0,
                                      bytes_accessed=bytes_accessed),
    )(x_cl, enc_ws, biases, mid_ws, dec_ws, out_ws)

    if N == 1:
        return out[:co].reshape(1, co, H, W)
    return out.reshape(co_p, N, H, W).transpose(1, 0, 2, 3)[:, :co]
